# baseline (speedup 1.0000x reference)
import functools

import jax
import jax.numpy as jnp
from jax import lax
from jax.experimental import pallas as pl
from jax.experimental.pallas import tpu as pltpu
from jax.experimental.pallas import tpu_sc as plsc

CONF_THRES = 0.25
IOU_THRES = 0.45
MAX_DET = 300
MAX_NMS = 2048
N_ANCH = 20000
CHUNK = 2000


def _score_body(pred_ref, packed_ref):
    pred = pred_ref[0]
    obj = pred[4:5, :]
    cls_conf = pred[5:, :] * obj
    conf = jnp.max(cls_conf, axis=0, keepdims=True)
    rows = jax.lax.broadcasted_iota(jnp.int32, cls_conf.shape, 0)
    j = jnp.min(jnp.where(cls_conf == conf, rows, 80), axis=0, keepdims=True)
    valid = (obj > CONF_THRES) & (conf > CONF_THRES)
    score = jnp.where(valid, conf, -1.0)
    xy = pred[0:2, :]
    wh = pred[2:4, :]
    half = wh / 2.0
    gidx = jax.lax.broadcasted_iota(jnp.int32, (1, N_ANCH), 1).astype(
        jnp.float32)
    packed_ref[0] = jnp.concatenate(
        [xy - half, xy + half, score, j.astype(jnp.float32), gidx, score],
        axis=0)


def _score_stage(predt):
    B = predt.shape[0]
    return pl.pallas_call(
        _score_body,
        grid=(B,),
        in_specs=[pl.BlockSpec((1, 85, N_ANCH), lambda b: (b, 0, 0))],
        out_specs=pl.BlockSpec((1, 8, N_ANCH), lambda b: (b, 0, 0)),
        out_shape=jax.ShapeDtypeStruct((B, 8, N_ANCH), jnp.float32),
    )(predt)


NMS_B = 128
NMS_NBLK = MAX_NMS // NMS_B


def _iou_pair(x1b, y1b, x2b, y2b, area_b, x1a, y1a, x2a, y2a, area_a):
    iw = jnp.minimum(x2b, x2a) - jnp.maximum(x1b, x1a)
    ih = jnp.minimum(y2b, y2a) - jnp.maximum(y1b, y1a)
    inter = jnp.clip(iw, 0.0) * jnp.clip(ih, 0.0)
    return inter / (area_b + area_a - inter + 1e-9)


SEL_S = 320


def _nms_body(sbr_ref, sbt_ref, det_ref, keep_ref):
    nb = sbr_ref.shape[0]
    B = NMS_B

    for k in range(NMS_NBLK):
        bb = sbt_ref[:, k * B:(k + 1) * B, :]
        x1b = bb[:, :, 0:1]
        y1b = bb[:, :, 1:2]
        x2b = bb[:, :, 2:3]
        y2b = bb[:, :, 3:4]
        area_b = (x2b - x1b) * (y2b - y1b)
        valid_b = bb[:, :, 4] > 0.0

        if k > 0:
            P = k * B
            x1a = sbr_ref[:, 0:1, :P]
            y1a = sbr_ref[:, 1:2, :P]
            x2a = sbr_ref[:, 2:3, :P]
            y2a = sbr_ref[:, 3:4, :P]
            area_a = (x2a - x1a) * (y2a - y1a)
            iou_p = _iou_pair(x1b, y1b, x2b, y2b, area_b,
                              x1a, y1a, x2a, y2a, area_a)
            keep_prev = keep_ref[:, :P]
            sup_prev = jnp.max(
                jnp.where((iou_p > IOU_THRES) & (keep_prev[:, None, :] > 0.0),
                          1.0, 0.0), axis=2)
            init_b = valid_b & (sup_prev == 0.0)
        else:
            init_b = valid_b

        x1bt = jnp.transpose(x1b, (0, 2, 1))
        y1bt = jnp.transpose(y1b, (0, 2, 1))
        x2bt = jnp.transpose(x2b, (0, 2, 1))
        y2bt = jnp.transpose(y2b, (0, 2, 1))
        area_bt = jnp.transpose(area_b, (0, 2, 1))
        ioub = _iou_pair(x1b, y1b, x2b, y2b, area_b,
                         x1bt, y1bt, x2bt, y2bt, area_bt)
        rows_p = jax.lax.broadcasted_iota(jnp.int32, (1, B, B), 1)
        cols_q = jax.lax.broadcasted_iota(jnp.int32, (1, B, B), 2)
        S = (ioub > IOU_THRES) & (rows_p < cols_q)

        Sf = jnp.where(S, 1.0, 0.0)
        init_f = jnp.where(init_b, 1.0, 0.0)

        def tstep(kb):
            sup = jnp.max(Sf * kb[:, :, None], axis=1)
            return init_f * (1.0 - sup)

        def fix_cond(c):
            _, changed = c
            return changed > 0

        def fix_body(c):
            kb, _ = c
            new = tstep(kb)
            return new, jnp.max(jnp.abs(new - kb)).astype(jnp.int32)

        kb1 = tstep(init_f)
        kb, _ = jax.lax.while_loop(
            fix_cond, fix_body,
            (kb1, jnp.max(jnp.abs(kb1 - init_f)).astype(jnp.int32)))
        keep_ref[:, k * B:(k + 1) * B] = kb

    keep = keep_ref[...]
    lanes = jax.lax.broadcasted_iota(jnp.int32, (1, MAX_NMS), 1)
    rank = keep
    for s in [1, 2, 4, 8, 16, 32, 64, 128, 256, 512, 1024]:
        rank = rank + jnp.where(lanes >= s, jnp.roll(rank, s, axis=1), 0.0)
    cols_s = jax.lax.broadcasted_iota(jnp.int32, (SEL_S, 1), 0).astype(jnp.float32)
    for b in range(nb):
        onehot = jnp.where(
            (rank[b:b + 1, :] == cols_s + 1.0) & (keep[b:b + 1, :] > 0.0),
            1.0, 0.0)
        det_ref[b] = jax.lax.dot_general(
            onehot, sbt_ref[b],
            dimension_numbers=(((1,), (0,)), ((), ())),
            preferred_element_type=jnp.float32,
            precision=jax.lax.Precision.HIGHEST)


def _nms_stage(sbr):
    B = sbr.shape[0]
    sbt = sbr.transpose(0, 2, 1)
    return pl.pallas_call(
        _nms_body,
        in_specs=[
            pl.BlockSpec(sbr.shape, lambda: (0, 0, 0)),
            pl.BlockSpec(sbt.shape, lambda: (0, 0, 0)),
        ],
        out_specs=pl.BlockSpec((B, SEL_S, 8), lambda: (0, 0, 0)),
        out_shape=jax.ShapeDtypeStruct((B, SEL_S, 8), jnp.float32),
        scratch_shapes=[pltpu.VMEM((B, MAX_NMS), jnp.float32)],
    )(sbr, sbt)


def _sc_gather(table, idx, D):
    B = idx.shape[0]
    info = plsc.get_sparse_core_info()
    nw = info.num_cores * info.num_subcores
    b_per_w = B // nw

    @functools.partial(
        pl.kernel,
        mesh=plsc.VectorSubcoreMesh(core_axis_name="c", subcore_axis_name="s"),
        out_type=jax.ShapeDtypeStruct((B, D), jnp.float32),
        compiler_params=pltpu.CompilerParams(use_tc_tiling_on_sc=False),
        scratch_types=[
            pltpu.VMEM((b_per_w,), jnp.int32),
            pltpu.VMEM((b_per_w, D), jnp.float32),
            pltpu.SemaphoreType.DMA,
        ],
    )
    def gather_kernel(table_hbm, idx_hbm, out_hbm, idx_v, rows_v, sem):
        wid = lax.axis_index("s") * info.num_cores + lax.axis_index("c")
        base = wid * b_per_w
        pltpu.sync_copy(idx_hbm.at[pl.ds(base, b_per_w)], idx_v)
        pltpu.async_copy(table_hbm.at[idx_v], rows_v, sem).wait()
        pltpu.sync_copy(rows_v, out_hbm.at[pl.ds(base, b_per_w)])

    return gather_kernel(table, idx)


def kernel(prediction, logits):
    predt = prediction.transpose(0, 2, 1)
    packed = _score_stage(predt)
    scores = packed[:, 4, :]
    top_scores, order = jax.lax.top_k(scores, MAX_NMS)
    sbr = jnp.take_along_axis(packed, order[:, None, :], axis=2)
    det = _nms_stage(sbr)
    ok = (det[:, :, 4:5] > 0.0).astype(jnp.float32)
    base = (jnp.arange(8, dtype=jnp.int32) * N_ANCH)[:, None]
    lidx = jnp.minimum(jnp.round(det[:, :, 6]).astype(jnp.int32),
                       N_ANCH - 1) + base
    out_log = _sc_gather(logits.reshape(8 * N_ANCH, 80),
                         lidx.reshape(-1), 80).reshape(8, SEL_S, 80)
    out_log = out_log[:, :MAX_DET] * ok[:, :MAX_DET]
    return jnp.concatenate([det[:, :MAX_DET, :6], out_log], axis=-1)


# speedup vs baseline: 1.5689x; 1.5689x over previous
import functools

import jax
import jax.numpy as jnp
from jax import lax
from jax.experimental import pallas as pl
from jax.experimental.pallas import tpu as pltpu
from jax.experimental.pallas import tpu_sc as plsc

CONF_THRES = 0.25
IOU_THRES = 0.45
MAX_DET = 300
MAX_NMS = 2048
N_ANCH = 20000
CHUNK = 2000


N_PAD = 20480
N_CHK = N_PAD // MAX_NMS


def _score_body(pred_ref, packed_ref):
    pred = pred_ref[0]
    obj = pred[4:5, :]
    cls_conf = pred[5:, :] * obj
    conf = jnp.max(cls_conf, axis=0, keepdims=True)
    rows = jax.lax.broadcasted_iota(jnp.int32, cls_conf.shape, 0)
    j = jnp.min(jnp.where(cls_conf == conf, rows, 80), axis=0, keepdims=True)
    valid = (obj > CONF_THRES) & (conf > CONF_THRES)
    score = jnp.where(valid, conf, -1.0)
    xy = pred[0:2, :]
    wh = pred[2:4, :]
    half = wh / 2.0
    gidx = jax.lax.broadcasted_iota(jnp.int32, (1, N_ANCH), 1).astype(
        jnp.float32)
    packed_ref[0, :, :N_ANCH] = jnp.concatenate(
        [xy - half, xy + half, score, j.astype(jnp.float32), gidx, score],
        axis=0)
    packed_ref[0, :, N_ANCH:] = jnp.full((8, N_PAD - N_ANCH), -1.0,
                                         jnp.float32)


def _score_stage(predt):
    B = predt.shape[0]
    return pl.pallas_call(
        _score_body,
        grid=(B,),
        in_specs=[pl.BlockSpec((1, 85, N_ANCH), lambda b: (b, 0, 0))],
        out_specs=pl.BlockSpec((1, 8, N_PAD), lambda b: (b, 0, 0)),
        out_shape=jax.ShapeDtypeStruct((B, 8, N_PAD), jnp.float32),
    )(predt)


def _beats(k, i, pk, pi):
    return (k > pk) | ((k == pk) & (i < pi))


def _ce(k, i, s, desc_block, lane):
    pk = jnp.where((lane & s) == 0, jnp.roll(k, -s, axis=-1),
                   jnp.roll(k, s, axis=-1))
    pi = jnp.where((lane & s) == 0, jnp.roll(i, -s, axis=-1),
                   jnp.roll(i, s, axis=-1))
    want_win = ((lane & s) == 0) == desc_block
    take_self = _beats(k, i, pk, pi) == want_win
    return jnp.where(take_self, k, pk), jnp.where(take_self, i, pi)


def _bitonic_merge(k, i, asc_rows, lane):
    desc = ~asc_rows
    for s in [1024, 512, 256, 128, 64, 32, 16, 8, 4, 2, 1]:
        k, i = _ce(k, i, s, desc, lane)
    return k, i


def _sort_body(scores_ref, order_ref):
    lane2 = jax.lax.broadcasted_iota(jnp.int32, (1, MAX_NMS), 1)
    lane3 = lane2[None]
    x = scores_ref[...]
    cidx = jax.lax.broadcasted_iota(jnp.int32, x.shape, 1)
    gi = cidx * MAX_NMS + jax.lax.broadcasted_iota(jnp.int32, x.shape, 2)
    k2 = x.reshape(8 * N_CHK, MAX_NMS)
    i2 = gi.reshape(8 * N_CHK, MAX_NMS)
    row_asc = (jax.lax.broadcasted_iota(jnp.int32, (8 * N_CHK, 1), 0)
               % N_CHK) % 2 == 1
    bs = 2
    while bs <= MAX_NMS:
        desc = ((lane2 & bs) == 0) ^ row_asc
        s = bs // 2
        while s >= 1:
            k2, i2 = _ce(k2, i2, s, desc, lane2)
            s //= 2
        bs *= 2
    k3 = k2.reshape(8, N_CHK, MAX_NMS)
    i3 = i2.reshape(8, N_CHK, MAX_NMS)

    def merge_level(ka, ia, kb, ib, asc_out):
        take = _beats(ka, ia, kb, ib)
        mk = jnp.where(take, ka, kb)
        mi = jnp.where(take, ia, ib)
        return _bitonic_merge(mk, mi, asc_out, lane3)

    c5 = jax.lax.broadcasted_iota(jnp.int32, (1, 5, 1), 1)
    asc5 = (c5 % 2 == 1) | (c5 == 4)
    k4 = k3.reshape(8, 5, 2, MAX_NMS)
    i4 = i3.reshape(8, 5, 2, MAX_NMS)
    mk, mi = merge_level(k4[:, :, 0], i4[:, :, 0],
                         k4[:, :, 1], i4[:, :, 1], asc5)
    c2 = jax.lax.broadcasted_iota(jnp.int32, (1, 2, 1), 1)
    mk4 = mk[:, 0:4].reshape(8, 2, 2, MAX_NMS)
    mi4 = mi[:, 0:4].reshape(8, 2, 2, MAX_NMS)
    pk, pi = merge_level(mk4[:, :, 0], mi4[:, :, 0],
                         mk4[:, :, 1], mi4[:, :, 1], c2 == 1)
    qk, qi = merge_level(pk[:, 0:1], pi[:, 0:1],
                         pk[:, 1:2], pi[:, 1:2],
                         jnp.zeros((1, 1, 1), jnp.bool_))
    fk, fi = merge_level(qk, qi, mk[:, 4:5], mi[:, 4:5],
                         jnp.zeros((1, 1, 1), jnp.bool_))
    order_ref[...] = fi.reshape(8, MAX_NMS)


def _sort_stage(scores3):
    return pl.pallas_call(
        _sort_body,
        in_specs=[pl.BlockSpec(scores3.shape, lambda: (0, 0, 0))],
        out_specs=pl.BlockSpec((8, MAX_NMS), lambda: (0, 0)),
        out_shape=jax.ShapeDtypeStruct((8, MAX_NMS), jnp.int32),
    )(scores3)


NMS_B = 128
NMS_NBLK = MAX_NMS // NMS_B


def _iou_pair(x1b, y1b, x2b, y2b, area_b, x1a, y1a, x2a, y2a, area_a):
    iw = jnp.minimum(x2b, x2a) - jnp.maximum(x1b, x1a)
    ih = jnp.minimum(y2b, y2a) - jnp.maximum(y1b, y1a)
    inter = jnp.clip(iw, 0.0) * jnp.clip(ih, 0.0)
    return inter / (area_b + area_a - inter + 1e-9)


SEL_S = 320


def _nms_body(sbr_ref, sbt_ref, det_ref, keep_ref):
    nb = sbr_ref.shape[0]
    B = NMS_B

    for k in range(NMS_NBLK):
        bb = sbt_ref[:, k * B:(k + 1) * B, :]
        x1b = bb[:, :, 0:1]
        y1b = bb[:, :, 1:2]
        x2b = bb[:, :, 2:3]
        y2b = bb[:, :, 3:4]
        area_b = (x2b - x1b) * (y2b - y1b)
        valid_b = bb[:, :, 4] > 0.0

        if k > 0:
            P = k * B
            x1a = sbr_ref[:, 0:1, :P]
            y1a = sbr_ref[:, 1:2, :P]
            x2a = sbr_ref[:, 2:3, :P]
            y2a = sbr_ref[:, 3:4, :P]
            area_a = (x2a - x1a) * (y2a - y1a)
            iou_p = _iou_pair(x1b, y1b, x2b, y2b, area_b,
                              x1a, y1a, x2a, y2a, area_a)
            keep_prev = keep_ref[:, :P]
            sup_prev = jnp.max(
                jnp.where((iou_p > IOU_THRES) & (keep_prev[:, None, :] > 0.0),
                          1.0, 0.0), axis=2)
            init_b = valid_b & (sup_prev == 0.0)
        else:
            init_b = valid_b

        x1bt = jnp.transpose(x1b, (0, 2, 1))
        y1bt = jnp.transpose(y1b, (0, 2, 1))
        x2bt = jnp.transpose(x2b, (0, 2, 1))
        y2bt = jnp.transpose(y2b, (0, 2, 1))
        area_bt = jnp.transpose(area_b, (0, 2, 1))
        ioub = _iou_pair(x1b, y1b, x2b, y2b, area_b,
                         x1bt, y1bt, x2bt, y2bt, area_bt)
        rows_p = jax.lax.broadcasted_iota(jnp.int32, (1, B, B), 1)
        cols_q = jax.lax.broadcasted_iota(jnp.int32, (1, B, B), 2)
        S = (ioub > IOU_THRES) & (rows_p < cols_q)

        Sf = jnp.where(S, 1.0, 0.0)
        init_f = jnp.where(init_b, 1.0, 0.0)

        def tstep(kb):
            sup = jnp.max(Sf * kb[:, :, None], axis=1)
            return init_f * (1.0 - sup)

        def fix_cond(c):
            _, changed = c
            return changed > 0

        def fix_body(c):
            kb, _ = c
            new = tstep(kb)
            return new, jnp.max(jnp.abs(new - kb)).astype(jnp.int32)

        kb1 = tstep(init_f)
        kb, _ = jax.lax.while_loop(
            fix_cond, fix_body,
            (kb1, jnp.max(jnp.abs(kb1 - init_f)).astype(jnp.int32)))
        keep_ref[:, k * B:(k + 1) * B] = kb

    keep = keep_ref[...]
    lanes = jax.lax.broadcasted_iota(jnp.int32, (1, MAX_NMS), 1)
    rank = keep
    for s in [1, 2, 4, 8, 16, 32, 64, 128, 256, 512, 1024]:
        rank = rank + jnp.where(lanes >= s, jnp.roll(rank, s, axis=1), 0.0)
    cols_s = jax.lax.broadcasted_iota(jnp.int32, (SEL_S, 1), 0).astype(jnp.float32)
    for b in range(nb):
        onehot = jnp.where(
            (rank[b:b + 1, :] == cols_s + 1.0) & (keep[b:b + 1, :] > 0.0),
            1.0, 0.0)
        det_ref[b] = jax.lax.dot_general(
            onehot, sbt_ref[b],
            dimension_numbers=(((1,), (0,)), ((), ())),
            preferred_element_type=jnp.float32,
            precision=jax.lax.Precision.HIGHEST)


def _nms_stage(sbr):
    B = sbr.shape[0]
    sbt = sbr.transpose(0, 2, 1)
    return pl.pallas_call(
        _nms_body,
        in_specs=[
            pl.BlockSpec(sbr.shape, lambda: (0, 0, 0)),
            pl.BlockSpec(sbt.shape, lambda: (0, 0, 0)),
        ],
        out_specs=pl.BlockSpec((B, SEL_S, 8), lambda: (0, 0, 0)),
        out_shape=jax.ShapeDtypeStruct((B, SEL_S, 8), jnp.float32),
        scratch_shapes=[pltpu.VMEM((B, MAX_NMS), jnp.float32)],
    )(sbr, sbt)


def kernel(prediction, logits):
    predt = prediction.transpose(0, 2, 1)
    packed = _score_stage(predt)
    scores3 = packed[:, 4, :].reshape(8, N_CHK, MAX_NMS)
    order = _sort_stage(scores3)
    sbr = jnp.take_along_axis(packed, order[:, None, :], axis=2)
    det = _nms_stage(sbr)[:, :MAX_DET]
    lidx = jnp.minimum(jnp.round(det[:, :, 6]).astype(jnp.int32), N_ANCH - 1)
    ok = (det[:, :, 4:5] > 0.0).astype(jnp.float32)
    out_log = jnp.take_along_axis(logits, lidx[:, :, None], axis=1) * ok
    return jnp.concatenate([det[:, :, :6], out_log], axis=-1)


# speedup vs baseline: 1.5709x; 1.0013x over previous
import functools

import jax
import jax.numpy as jnp
from jax import lax
from jax.experimental import pallas as pl
from jax.experimental.pallas import tpu as pltpu
from jax.experimental.pallas import tpu_sc as plsc

CONF_THRES = 0.25
IOU_THRES = 0.45
MAX_DET = 300
MAX_NMS = 2048
N_ANCH = 20000
CHUNK = 2000


N_PAD = 20480
N_CHK = N_PAD // MAX_NMS


def _score_body(pred_ref, packed_ref):
    pred = pred_ref[0]
    obj = pred[4:5, :]
    cls_conf = pred[5:, :] * obj
    conf = jnp.max(cls_conf, axis=0, keepdims=True)
    rows = jax.lax.broadcasted_iota(jnp.int32, cls_conf.shape, 0)
    j = jnp.min(jnp.where(cls_conf == conf, rows, 80), axis=0, keepdims=True)
    valid = (obj > CONF_THRES) & (conf > CONF_THRES)
    score = jnp.where(valid, conf, -1.0)
    xy = pred[0:2, :]
    wh = pred[2:4, :]
    half = wh / 2.0
    gidx = jax.lax.broadcasted_iota(jnp.int32, (1, N_ANCH), 1).astype(
        jnp.float32)
    packed_ref[0, :, :N_ANCH] = jnp.concatenate(
        [xy - half, xy + half, score, j.astype(jnp.float32), gidx, score],
        axis=0)
    packed_ref[0, :, N_ANCH:] = jnp.full((8, N_PAD - N_ANCH), -1.0,
                                         jnp.float32)


def _score_stage(predt):
    B = predt.shape[0]
    return pl.pallas_call(
        _score_body,
        grid=(B,),
        in_specs=[pl.BlockSpec((1, 85, N_ANCH), lambda b: (b, 0, 0))],
        out_specs=pl.BlockSpec((1, 8, N_PAD), lambda b: (b, 0, 0)),
        out_shape=jax.ShapeDtypeStruct((B, 8, N_PAD), jnp.float32),
    )(predt)


def _beats(k, i, pk, pi):
    return (k > pk) | ((k == pk) & (i < pi))


def _ce(k, i, s, desc_block, lane):
    pk = jnp.where((lane & s) == 0, jnp.roll(k, -s, axis=-1),
                   jnp.roll(k, s, axis=-1))
    pi = jnp.where((lane & s) == 0, jnp.roll(i, -s, axis=-1),
                   jnp.roll(i, s, axis=-1))
    want_win = ((lane & s) == 0) == desc_block
    take_self = _beats(k, i, pk, pi) == want_win
    return jnp.where(take_self, k, pk), jnp.where(take_self, i, pi)


def _bitonic_merge(k, i, asc_rows, lane):
    desc = ~asc_rows
    for s in [1024, 512, 256, 128, 64, 32, 16, 8, 4, 2, 1]:
        k, i = _ce(k, i, s, desc, lane)
    return k, i


def _sort_body(scores_ref, order_ref):
    lane2 = jax.lax.broadcasted_iota(jnp.int32, (1, MAX_NMS), 1)
    lane3 = lane2[None]
    x = scores_ref[...]
    cidx = jax.lax.broadcasted_iota(jnp.int32, x.shape, 1)
    gi = cidx * MAX_NMS + jax.lax.broadcasted_iota(jnp.int32, x.shape, 2)
    k2 = x.reshape(8 * N_CHK, MAX_NMS)
    i2 = gi.reshape(8 * N_CHK, MAX_NMS)
    row_asc = (jax.lax.broadcasted_iota(jnp.int32, (8 * N_CHK, 1), 0)
               % N_CHK) % 2 == 1
    bs = 2
    while bs <= MAX_NMS:
        desc = ((lane2 & bs) == 0) ^ row_asc
        s = bs // 2
        while s >= 1:
            k2, i2 = _ce(k2, i2, s, desc, lane2)
            s //= 2
        bs *= 2
    k3 = k2.reshape(8, N_CHK, MAX_NMS)
    i3 = i2.reshape(8, N_CHK, MAX_NMS)

    def merge_level(ka, ia, kb, ib, asc_out):
        take = _beats(ka, ia, kb, ib)
        mk = jnp.where(take, ka, kb)
        mi = jnp.where(take, ia, ib)
        return _bitonic_merge(mk, mi, asc_out, lane3)

    c5 = jax.lax.broadcasted_iota(jnp.int32, (1, 5, 1), 1)
    asc5 = (c5 % 2 == 1) | (c5 == 4)
    k4 = k3.reshape(8, 5, 2, MAX_NMS)
    i4 = i3.reshape(8, 5, 2, MAX_NMS)
    mk, mi = merge_level(k4[:, :, 0], i4[:, :, 0],
                         k4[:, :, 1], i4[:, :, 1], asc5)
    c2 = jax.lax.broadcasted_iota(jnp.int32, (1, 2, 1), 1)
    mk4 = mk[:, 0:4].reshape(8, 2, 2, MAX_NMS)
    mi4 = mi[:, 0:4].reshape(8, 2, 2, MAX_NMS)
    pk, pi = merge_level(mk4[:, :, 0], mi4[:, :, 0],
                         mk4[:, :, 1], mi4[:, :, 1], c2 == 1)
    qk, qi = merge_level(pk[:, 0:1], pi[:, 0:1],
                         pk[:, 1:2], pi[:, 1:2],
                         jnp.zeros((1, 1, 1), jnp.bool_))
    fk, fi = merge_level(qk, qi, mk[:, 4:5], mi[:, 4:5],
                         jnp.zeros((1, 1, 1), jnp.bool_))
    order_ref[...] = fi.reshape(8, MAX_NMS)


def _sort_stage(scores3):
    return pl.pallas_call(
        _sort_body,
        in_specs=[pl.BlockSpec(scores3.shape, lambda: (0, 0, 0))],
        out_specs=pl.BlockSpec((8, MAX_NMS), lambda: (0, 0)),
        out_shape=jax.ShapeDtypeStruct((8, MAX_NMS), jnp.int32),
    )(scores3)


NMS_B = 256
NMS_NBLK = MAX_NMS // NMS_B


def _iou_pair(x1b, y1b, x2b, y2b, area_b, x1a, y1a, x2a, y2a, area_a):
    iw = jnp.minimum(x2b, x2a) - jnp.maximum(x1b, x1a)
    ih = jnp.minimum(y2b, y2a) - jnp.maximum(y1b, y1a)
    inter = jnp.clip(iw, 0.0) * jnp.clip(ih, 0.0)
    return inter / (area_b + area_a - inter + 1e-9)


SEL_S = 320


def _nms_body(sbr_ref, sbt_ref, det_ref, keep_ref):
    nb = sbr_ref.shape[0]
    B = NMS_B

    for k in range(NMS_NBLK):
        bb = sbt_ref[:, k * B:(k + 1) * B, :]
        x1b = bb[:, :, 0:1]
        y1b = bb[:, :, 1:2]
        x2b = bb[:, :, 2:3]
        y2b = bb[:, :, 3:4]
        area_b = (x2b - x1b) * (y2b - y1b)
        valid_b = bb[:, :, 4] > 0.0

        if k > 0:
            P = k * B
            x1a = sbr_ref[:, 0:1, :P]
            y1a = sbr_ref[:, 1:2, :P]
            x2a = sbr_ref[:, 2:3, :P]
            y2a = sbr_ref[:, 3:4, :P]
            area_a = (x2a - x1a) * (y2a - y1a)
            iou_p = _iou_pair(x1b, y1b, x2b, y2b, area_b,
                              x1a, y1a, x2a, y2a, area_a)
            keep_prev = keep_ref[:, :P]
            sup_prev = jnp.max(
                jnp.where((iou_p > IOU_THRES) & (keep_prev[:, None, :] > 0.0),
                          1.0, 0.0), axis=2)
            init_b = valid_b & (sup_prev == 0.0)
        else:
            init_b = valid_b

        x1bt = jnp.transpose(x1b, (0, 2, 1))
        y1bt = jnp.transpose(y1b, (0, 2, 1))
        x2bt = jnp.transpose(x2b, (0, 2, 1))
        y2bt = jnp.transpose(y2b, (0, 2, 1))
        area_bt = jnp.transpose(area_b, (0, 2, 1))
        ioub = _iou_pair(x1b, y1b, x2b, y2b, area_b,
                         x1bt, y1bt, x2bt, y2bt, area_bt)
        rows_p = jax.lax.broadcasted_iota(jnp.int32, (1, B, B), 1)
        cols_q = jax.lax.broadcasted_iota(jnp.int32, (1, B, B), 2)
        S = (ioub > IOU_THRES) & (rows_p < cols_q)

        Sf = jnp.where(S, 1.0, 0.0)
        init_f = jnp.where(init_b, 1.0, 0.0)

        def tstep(kb):
            sup = jnp.max(Sf * kb[:, :, None], axis=1)
            return init_f * (1.0 - sup)

        def fix_cond(c):
            _, changed = c
            return changed > 0

        def fix_body(c):
            kb, _ = c
            new = tstep(kb)
            return new, jnp.max(jnp.abs(new - kb)).astype(jnp.int32)

        kb1 = tstep(init_f)
        kb, _ = jax.lax.while_loop(
            fix_cond, fix_body,
            (kb1, jnp.max(jnp.abs(kb1 - init_f)).astype(jnp.int32)))
        keep_ref[:, k * B:(k + 1) * B] = kb

    keep = keep_ref[...]
    lanes = jax.lax.broadcasted_iota(jnp.int32, (1, MAX_NMS), 1)
    rank = keep
    for s in [1, 2, 4, 8, 16, 32, 64, 128, 256, 512, 1024]:
        rank = rank + jnp.where(lanes >= s, jnp.roll(rank, s, axis=1), 0.0)
    cols_s = jax.lax.broadcasted_iota(jnp.int32, (SEL_S, 1), 0).astype(jnp.float32)
    for b in range(nb):
        onehot = jnp.where(
            (rank[b:b + 1, :] == cols_s + 1.0) & (keep[b:b + 1, :] > 0.0),
            1.0, 0.0)
        det_ref[b] = jax.lax.dot_general(
            onehot, sbt_ref[b],
            dimension_numbers=(((1,), (0,)), ((), ())),
            preferred_element_type=jnp.float32,
            precision=jax.lax.Precision.HIGHEST)


def _nms_stage(sbr):
    B = sbr.shape[0]
    sbt = sbr.transpose(0, 2, 1)
    return pl.pallas_call(
        _nms_body,
        in_specs=[
            pl.BlockSpec(sbr.shape, lambda: (0, 0, 0)),
            pl.BlockSpec(sbt.shape, lambda: (0, 0, 0)),
        ],
        out_specs=pl.BlockSpec((B, SEL_S, 8), lambda: (0, 0, 0)),
        out_shape=jax.ShapeDtypeStruct((B, SEL_S, 8), jnp.float32),
        scratch_shapes=[pltpu.VMEM((B, MAX_NMS), jnp.float32)],
    )(sbr, sbt)


def kernel(prediction, logits):
    predt = prediction.transpose(0, 2, 1)
    packed = _score_stage(predt)
    scores3 = packed[:, 4, :].reshape(8, N_CHK, MAX_NMS)
    order = _sort_stage(scores3)
    sbr = jnp.take_along_axis(packed, order[:, None, :], axis=2)
    det = _nms_stage(sbr)[:, :MAX_DET]
    lidx = jnp.minimum(jnp.round(det[:, :, 6]).astype(jnp.int32), N_ANCH - 1)
    ok = (det[:, :, 4:5] > 0.0).astype(jnp.float32)
    out_log = jnp.take_along_axis(logits, lidx[:, :, None], axis=1) * ok
    return jnp.concatenate([det[:, :, :6], out_log], axis=-1)
